# baseline (speedup 1.0000x reference)
import functools

import jax
import jax.numpy as jnp
from jax import lax
from jax.experimental import pallas as pl
from jax.experimental.pallas import tpu as pltpu
from jax.experimental.pallas import tpu_sc as plsc

_VOCAB = 100000
_EMBED = 128
_BATCH = 4096
_SEQLEN = 200
_NCLASS = 5

_NC = 2
_NS = 16
_LANES = 16
_NW = _NC * _NS

_EPT = _BATCH // _NW
_NBUF = 2
_C1 = 104
_C2 = _SEQLEN - _C1


_PACK = 128 // _LANES
_VROWS = _VOCAB // _PACK


def _proj_body(t_ref, w_ref, o_ref):
    acc = jnp.dot(t_ref[:, 0, :], w_ref[0],
                  preferred_element_type=jnp.float32)
    for g in range(1, _PACK):
        acc = acc + jnp.dot(t_ref[:, g, :], w_ref[g],
                            preferred_element_type=jnp.float32)
    o_ref[...] = acc


def _project(table3, w_t):
    bm = 1000
    return pl.pallas_call(
        _proj_body,
        grid=(pl.cdiv(_VROWS, bm),),
        in_specs=[
            pl.BlockSpec((bm, _PACK, _EMBED), lambda i: (i, 0, 0)),
            pl.BlockSpec((_PACK, _EMBED, 128), lambda i: (0, 0, 0)),
        ],
        out_specs=pl.BlockSpec((bm, 128), lambda i: (i, 0)),
        out_shape=jax.ShapeDtypeStruct((_VROWS, 128), jnp.float32),
    )(table3, w_t)


def _sc_body(idx_hbm, proj_hbm, bias_hbm, out_hbm,
             idx_v, rows_v, out_v, bias_v, *sems):
    wid = lax.axis_index("s") * _NC + lax.axis_index("c")
    base = wid * _EPT

    pltpu.sync_copy(idx_hbm.at[pl.ds(base * _SEQLEN, _EPT * _SEQLEN)], idx_v)
    pltpu.sync_copy(bias_hbm, bias_v)
    bias = bias_v[...]

    def fire(e, slot):
        off = pl.multiple_of(e * _SEQLEN, 8)
        pltpu.async_copy(proj_hbm.at[idx_v.at[pl.ds(off, _C1)]],
                         rows_v.at[slot, pl.ds(0, _C1)], sems[slot])
        off2 = pl.multiple_of(e * _SEQLEN + _C1, 8)
        pltpu.async_copy(proj_hbm.at[idx_v.at[pl.ds(off2, _C2)]],
                         rows_v.at[slot, pl.ds(_C1, _C2)], sems[slot])

    def drain(slot):
        pltpu.make_async_copy(proj_hbm.at[idx_v.at[pl.ds(0, _C1)]],
                              rows_v.at[slot, pl.ds(0, _C1)],
                              sems[slot]).wait()
        pltpu.make_async_copy(proj_hbm.at[idx_v.at[pl.ds(0, _C2)]],
                              rows_v.at[slot, pl.ds(_C1, _C2)],
                              sems[slot]).wait()

    def reduce(slot):
        def step(i, accs):
            return tuple(accs[a] + rows_v[slot, i * 8 + a, :]
                         for a in range(8))
        accs = lax.fori_loop(
            1, _SEQLEN // 8, step,
            tuple(rows_v[slot, a, :] for a in range(8)))
        return (((accs[0] + accs[1]) + (accs[2] + accs[3]))
                + ((accs[4] + accs[5]) + (accs[6] + accs[7])))

    for s in range(_NBUF):
        fire(s, s)

    def body(g, carry):
        for s in range(_NBUF):
            e = g * _NBUF + s
            drain(s)
            out_v[e, :] = reduce(s) + bias
            nxt = e + _NBUF

            @pl.when(nxt < _EPT)
            def _():
                fire(nxt, s)
        return carry

    lax.fori_loop(0, _EPT // _NBUF, body, 0)

    pltpu.sync_copy(out_v, out_hbm.at[pl.ds(base, _EPT)])


_sc_pool = functools.partial(
    pl.kernel,
    out_type=jax.ShapeDtypeStruct((_BATCH, _LANES), jnp.float32),
    mesh=plsc.VectorSubcoreMesh(core_axis_name="c", subcore_axis_name="s",
                                num_cores=_NC, num_subcores=_NS),
    scratch_types=[
        pltpu.VMEM((_EPT * _SEQLEN,), jnp.int32),
        pltpu.VMEM((_NBUF, _SEQLEN, _LANES), jnp.float32),
        pltpu.VMEM((_EPT, _LANES), jnp.float32),
        pltpu.VMEM((_LANES,), jnp.float32),
    ] + [pltpu.SemaphoreType.DMA] * _NBUF,
    compiler_params=pltpu.CompilerParams(use_tc_tiling_on_sc=False),
)(_sc_body)


def kernel(inputs, table, W, b):
    idx = inputs.astype(jnp.int32).reshape(_BATCH * _SEQLEN)
    w_t = jnp.zeros((_LANES, _EMBED), jnp.float32).at[:_NCLASS].set(
        W * jnp.float32(1.0 / _SEQLEN)).T
    bias = jnp.zeros((_LANES,), jnp.float32).at[:_NCLASS].set(b)
    w_all = jnp.zeros((_PACK, _EMBED, 128), jnp.float32)
    for g in range(_PACK):
        w_all = w_all.at[g, :, g * _LANES:(g + 1) * _LANES].set(w_t)
    proj = _project(table.reshape(_VROWS, _PACK, _EMBED), w_all)
    out16 = _sc_pool(idx, proj.reshape(_VOCAB, _LANES), bias)
    return out16[:, :_NCLASS]


# speedup vs baseline: 1.3099x; 1.3099x over previous
import functools

import jax
import jax.numpy as jnp
from jax import lax
from jax.experimental import pallas as pl
from jax.experimental.pallas import tpu as pltpu
from jax.experimental.pallas import tpu_sc as plsc

_VOCAB = 100000
_EMBED = 128
_BATCH = 4096
_SEQLEN = 200
_NCLASS = 5

_NC = 2
_NS = 16
_LANES = 16
_NW = _NC * _NS

_EPT = _BATCH // _NW
_NBUF = 4
_C1 = 104
_C2 = _SEQLEN - _C1


_PACK = 128 // _LANES
_VROWS = _VOCAB // _PACK


_BM = 1000


def _proj_body(t_ref, w_ref, o_ref):
    acc = jnp.dot(t_ref[:, 0, :], w_ref[0],
                  preferred_element_type=jnp.float32)
    for g in range(1, _PACK):
        acc = acc + jnp.dot(t_ref[:, g, :], w_ref[g],
                            preferred_element_type=jnp.float32)
    o_ref[...] = acc


def _project(table3, w_t):
    return pl.pallas_call(
        _proj_body,
        grid=(pl.cdiv(_VROWS, _BM),),
        in_specs=[
            pl.BlockSpec((_BM, _PACK, _EMBED), lambda i: (i, 0, 0)),
            pl.BlockSpec((_PACK, _EMBED, 128), lambda i: (0, 0, 0)),
        ],
        out_specs=pl.BlockSpec((_BM, 128), lambda i: (i, 0)),
        out_shape=jax.ShapeDtypeStruct((_VROWS, 128), jnp.float32),
    )(table3, w_t)


def _sc_body(idx_hbm, proj_hbm, bias_hbm, out_hbm,
             idx_v, rows_v, out_v, bias_v, *sems):
    wid = lax.axis_index("s") * _NC + lax.axis_index("c")
    base = wid * _EPT

    pltpu.sync_copy(idx_hbm.at[pl.ds(base * _SEQLEN, _EPT * _SEQLEN)], idx_v)
    pltpu.sync_copy(bias_hbm, bias_v)
    bias = bias_v[...]

    def fire(e, slot):
        off = pl.multiple_of(e * _SEQLEN, 8)
        pltpu.async_copy(proj_hbm.at[idx_v.at[pl.ds(off, _C1)]],
                         rows_v.at[slot, pl.ds(0, _C1)], sems[slot])
        off2 = pl.multiple_of(e * _SEQLEN + _C1, 8)
        pltpu.async_copy(proj_hbm.at[idx_v.at[pl.ds(off2, _C2)]],
                         rows_v.at[slot, pl.ds(_C1, _C2)], sems[slot])

    def drain(slot):
        pltpu.make_async_copy(proj_hbm.at[idx_v.at[pl.ds(0, _C1)]],
                              rows_v.at[slot, pl.ds(0, _C1)],
                              sems[slot]).wait()
        pltpu.make_async_copy(proj_hbm.at[idx_v.at[pl.ds(0, _C2)]],
                              rows_v.at[slot, pl.ds(_C1, _C2)],
                              sems[slot]).wait()

    def reduce(slot):
        def step(i, accs):
            return tuple(accs[a] + rows_v[slot, i * 8 + a, :]
                         for a in range(8))
        accs = lax.fori_loop(
            1, _SEQLEN // 8, step,
            tuple(rows_v[slot, a, :] for a in range(8)))
        return (((accs[0] + accs[1]) + (accs[2] + accs[3]))
                + ((accs[4] + accs[5]) + (accs[6] + accs[7])))

    for s in range(_NBUF):
        fire(s, s)

    def body(g, carry):
        for s in range(_NBUF):
            e = g * _NBUF + s
            drain(s)
            out_v[e, :] = reduce(s) + bias
            nxt = e + _NBUF

            @pl.when(nxt < _EPT)
            def _():
                fire(nxt, s)
        return carry

    lax.fori_loop(0, _EPT // _NBUF, body, 0)

    pltpu.sync_copy(out_v, out_hbm.at[pl.ds(base, _EPT)])


_sc_pool = functools.partial(
    pl.kernel,
    out_type=jax.ShapeDtypeStruct((_BATCH, _LANES), jnp.float32),
    mesh=plsc.VectorSubcoreMesh(core_axis_name="c", subcore_axis_name="s",
                                num_cores=_NC, num_subcores=_NS),
    scratch_types=[
        pltpu.VMEM((_EPT * _SEQLEN,), jnp.int32),
        pltpu.VMEM((_NBUF, _SEQLEN, _LANES), jnp.float32),
        pltpu.VMEM((_EPT, _LANES), jnp.float32),
        pltpu.VMEM((_LANES,), jnp.float32),
    ] + [pltpu.SemaphoreType.DMA] * _NBUF,
    compiler_params=pltpu.CompilerParams(use_tc_tiling_on_sc=False),
)(_sc_body)


def kernel(inputs, table, W, b):
    idx = inputs.astype(jnp.int32).reshape(_BATCH * _SEQLEN)
    w_t = jnp.zeros((_LANES, _EMBED), jnp.float32).at[:_NCLASS].set(
        W * jnp.float32(1.0 / _SEQLEN)).T
    bias = jnp.zeros((_LANES,), jnp.float32).at[:_NCLASS].set(b)
    w_all = (jnp.eye(_PACK, dtype=jnp.float32)[:, None, :, None]
             * w_t[None, :, None, :]).reshape(_PACK, _EMBED, 128)
    proj = _project(table.reshape(_VROWS, _PACK, _EMBED), w_all)
    out16 = _sc_pool(idx, proj.reshape(_VOCAB, _LANES), bias)
    return out16[:, :_NCLASS]
